# initial kernel scaffold (unmeasured)
import jax
import jax.numpy as jnp
from jax import lax
from jax.experimental import pallas as pl
from jax.experimental.pallas import tpu as pltpu

N_DEV = 16
SQ = 512
SKV = 2048
D = 1024
HQ = 8
DH = 128
SCALE = 0.08838834764831843


def kernel(x, Wq, Wo, K_ext, V_ext):
    x2 = x.reshape(SQ, D)
    K = K_ext.reshape(SKV, HQ, DH)
    V = V_ext.reshape(SKV, HQ, DH)

    def body(x_ref, wq_ref, wo_ref, k_ref, v_ref, out_ref,
             q_ref, acc_ref, stats_ref, comm_ref, cstats_ref,
             send_sems, recv_sems, ssend_sems, srecv_sems):
        my = lax.axis_index("i")
        left = lax.rem(my + N_DEV - 1, N_DEV)
        right = lax.rem(my + 1, N_DEV)

        barrier_sem = pltpu.get_barrier_semaphore()
        pl.semaphore_signal(barrier_sem, inc=1, device_id=(left,),
                            device_id_type=pl.DeviceIdType.MESH)
        pl.semaphore_signal(barrier_sem, inc=1, device_id=(right,),
                            device_id_type=pl.DeviceIdType.MESH)
        pl.semaphore_wait(barrier_sem, 2)

        q_ref[...] = jnp.dot(x_ref[...], wq_ref[...],
                             preferred_element_type=jnp.float32)

        for h in range(HQ):
            qh = q_ref[:, h * DH:(h + 1) * DH]
            kh = k_ref[:, h, :]
            s = lax.dot_general(qh, kh, (((1,), (1,)), ((), ())),
                                preferred_element_type=jnp.float32) * SCALE
            m = jnp.max(s, axis=1, keepdims=True)
            p = jnp.exp(s - m)
            l = jnp.sum(p, axis=1, keepdims=True)
            o = jnp.dot(p, v_ref[:, h, :], preferred_element_type=jnp.float32)
            acc_ref[:, h * DH:(h + 1) * DH] = o
            stats_ref[:, h:h + 1] = m
            stats_ref[:, HQ + h:HQ + h + 1] = l

        comm_ref[0] = acc_ref[...]
        cstats_ref[0] = stats_ref[...]

        for hop in range(N_DEV - 1):
            ss = hop % 2
            rs = (hop + 1) % 2
            r_out = pltpu.make_async_remote_copy(
                src_ref=comm_ref.at[ss], dst_ref=comm_ref.at[rs],
                send_sem=send_sems.at[ss], recv_sem=recv_sems.at[rs],
                device_id=(right,), device_id_type=pl.DeviceIdType.MESH)
            r_st = pltpu.make_async_remote_copy(
                src_ref=cstats_ref.at[ss], dst_ref=cstats_ref.at[rs],
                send_sem=ssend_sems.at[ss], recv_sem=srecv_sems.at[rs],
                device_id=(right,), device_id_type=pl.DeviceIdType.MESH)
            r_out.start()
            r_st.start()
            r_out.wait()
            r_st.wait()

            rm = cstats_ref[rs, :, 0:HQ]
            rl = cstats_ref[rs, :, HQ:2 * HQ]
            am = stats_ref[:, 0:HQ]
            al = stats_ref[:, HQ:2 * HQ]
            m_new = jnp.maximum(am, rm)
            a1 = jnp.exp(am - m_new)
            a2 = jnp.exp(rm - m_new)
            stats_ref[:, 0:HQ] = m_new
            stats_ref[:, HQ:2 * HQ] = al * a1 + rl * a2
            for h in range(HQ):
                acc_ref[:, h * DH:(h + 1) * DH] = (
                    acc_ref[:, h * DH:(h + 1) * DH] * a1[:, h:h + 1]
                    + comm_ref[rs, :, h * DH:(h + 1) * DH] * a2[:, h:h + 1])

        for h in range(HQ):
            q_ref[:, h * DH:(h + 1) * DH] = (
                acc_ref[:, h * DH:(h + 1) * DH]
                / stats_ref[:, HQ + h:HQ + h + 1])
        out_ref[...] = jnp.dot(q_ref[...], wo_ref[...],
                               preferred_element_type=jnp.float32)

    out = pl.pallas_call(
        body,
        out_shape=jax.ShapeDtypeStruct((SQ, D), jnp.float32),
        in_specs=[pl.BlockSpec(memory_space=pltpu.VMEM)] * 5,
        out_specs=pl.BlockSpec(memory_space=pltpu.VMEM),
        scratch_shapes=[
            pltpu.VMEM((SQ, D), jnp.float32),
            pltpu.VMEM((SQ, D), jnp.float32),
            pltpu.VMEM((SQ, 2 * HQ), jnp.float32),
            pltpu.VMEM((2, SQ, D), jnp.float32),
            pltpu.VMEM((2, SQ, 2 * HQ), jnp.float32),
            pltpu.SemaphoreType.DMA((2,)),
            pltpu.SemaphoreType.DMA((2,)),
            pltpu.SemaphoreType.DMA((2,)),
            pltpu.SemaphoreType.DMA((2,)),
        ],
        compiler_params=pltpu.CompilerParams(collective_id=0),
    )(x2, Wq, Wo, K, V)
    return out.reshape(1, SQ, D)


# baseline (device time: 468784 ns/iter reference)
import jax
import jax.numpy as jnp
from jax import lax
from jax.experimental import pallas as pl
from jax.experimental.pallas import tpu as pltpu

N_DEV = 16
SQ = 512
SKV = 2048
D = 1024
HQ = 8
DH = 128
KBLK = 512
SCALE = 0.08838834764831843


def kernel(x, Wq, Wo, K_ext, V_ext):
    x2 = x.reshape(SQ, D)
    K = K_ext.reshape(SKV, HQ, DH)
    V = V_ext.reshape(SKV, HQ, DH)

    def body(x_ref, wq_ref, wo_ref, k_hbm, v_hbm, out_ref,
             q_ref, acc_ref, stats_ref, comm_ref, cstats_ref,
             kh_ref, vh_ref, copy_sems,
             send_sems, recv_sems, ssend_sems, srecv_sems):
        my = lax.axis_index("i")
        left = lax.rem(my + N_DEV - 1, N_DEV)
        right = lax.rem(my + 1, N_DEV)

        barrier_sem = pltpu.get_barrier_semaphore()
        pl.semaphore_signal(barrier_sem, inc=1, device_id=(left,),
                            device_id_type=pl.DeviceIdType.MESH)
        pl.semaphore_signal(barrier_sem, inc=1, device_id=(right,),
                            device_id_type=pl.DeviceIdType.MESH)
        pl.semaphore_wait(barrier_sem, 2)

        q_ref[...] = jnp.dot(x_ref[...], wq_ref[...],
                             preferred_element_type=jnp.float32)

        for h in range(HQ):
            ck = pltpu.make_async_copy(k_hbm.at[:, h, :], kh_ref,
                                       copy_sems.at[0])
            cv = pltpu.make_async_copy(v_hbm.at[:, h, :], vh_ref,
                                       copy_sems.at[1])
            ck.start()
            cv.start()
            ck.wait()
            cv.wait()

            qh = q_ref[:, h * DH:(h + 1) * DH]
            m = None
            l = None
            o = None
            for b in range(SKV // KBLK):
                kb = kh_ref[b * KBLK:(b + 1) * KBLK, :]
                vb = vh_ref[b * KBLK:(b + 1) * KBLK, :]
                s = lax.dot_general(qh, kb, (((1,), (1,)), ((), ())),
                                    preferred_element_type=jnp.float32) * SCALE
                mb = jnp.max(s, axis=1, keepdims=True)
                if b == 0:
                    m = mb
                    p = jnp.exp(s - m)
                    l = jnp.sum(p, axis=1, keepdims=True)
                    o = jnp.dot(p, vb, preferred_element_type=jnp.float32)
                else:
                    m_new = jnp.maximum(m, mb)
                    a1 = jnp.exp(m - m_new)
                    p = jnp.exp(s - m_new)
                    l = l * a1 + jnp.sum(p, axis=1, keepdims=True)
                    o = o * a1 + jnp.dot(p, vb,
                                         preferred_element_type=jnp.float32)
                    m = m_new
            acc_ref[:, h * DH:(h + 1) * DH] = o
            stats_ref[:, h:h + 1] = m
            stats_ref[:, HQ + h:HQ + h + 1] = l

        comm_ref[0] = acc_ref[...]
        cstats_ref[0] = stats_ref[...]

        for hop in range(N_DEV - 1):
            ss = hop % 2
            rs = (hop + 1) % 2
            r_out = pltpu.make_async_remote_copy(
                src_ref=comm_ref.at[ss], dst_ref=comm_ref.at[rs],
                send_sem=send_sems.at[ss], recv_sem=recv_sems.at[rs],
                device_id=(right,), device_id_type=pl.DeviceIdType.MESH)
            r_st = pltpu.make_async_remote_copy(
                src_ref=cstats_ref.at[ss], dst_ref=cstats_ref.at[rs],
                send_sem=ssend_sems.at[ss], recv_sem=srecv_sems.at[rs],
                device_id=(right,), device_id_type=pl.DeviceIdType.MESH)
            r_out.start()
            r_st.start()
            r_out.wait()
            r_st.wait()

            rm = cstats_ref[rs, :, 0:HQ]
            rl = cstats_ref[rs, :, HQ:2 * HQ]
            am = stats_ref[:, 0:HQ]
            al = stats_ref[:, HQ:2 * HQ]
            m_new = jnp.maximum(am, rm)
            a1 = jnp.exp(am - m_new)
            a2 = jnp.exp(rm - m_new)
            stats_ref[:, 0:HQ] = m_new
            stats_ref[:, HQ:2 * HQ] = al * a1 + rl * a2
            for h in range(HQ):
                acc_ref[:, h * DH:(h + 1) * DH] = (
                    acc_ref[:, h * DH:(h + 1) * DH] * a1[:, h:h + 1]
                    + comm_ref[rs, :, h * DH:(h + 1) * DH] * a2[:, h:h + 1])

        for h in range(HQ):
            q_ref[:, h * DH:(h + 1) * DH] = (
                acc_ref[:, h * DH:(h + 1) * DH]
                / stats_ref[:, HQ + h:HQ + h + 1])
        out_ref[...] = jnp.dot(q_ref[...], wo_ref[...],
                               preferred_element_type=jnp.float32)

    out = pl.pallas_call(
        body,
        out_shape=jax.ShapeDtypeStruct((SQ, D), jnp.float32),
        in_specs=[
            pl.BlockSpec(memory_space=pltpu.VMEM),
            pl.BlockSpec(memory_space=pltpu.VMEM),
            pl.BlockSpec(memory_space=pltpu.VMEM),
            pl.BlockSpec(memory_space=pl.ANY),
            pl.BlockSpec(memory_space=pl.ANY),
        ],
        out_specs=pl.BlockSpec(memory_space=pltpu.VMEM),
        scratch_shapes=[
            pltpu.VMEM((SQ, D), jnp.float32),
            pltpu.VMEM((SQ, D), jnp.float32),
            pltpu.VMEM((SQ, 2 * HQ), jnp.float32),
            pltpu.VMEM((2, SQ, D), jnp.float32),
            pltpu.VMEM((2, SQ, 2 * HQ), jnp.float32),
            pltpu.VMEM((SKV, DH), jnp.float32),
            pltpu.VMEM((SKV, DH), jnp.float32),
            pltpu.SemaphoreType.DMA((2,)),
            pltpu.SemaphoreType.DMA((2,)),
            pltpu.SemaphoreType.DMA((2,)),
            pltpu.SemaphoreType.DMA((2,)),
            pltpu.SemaphoreType.DMA((2,)),
        ],
        compiler_params=pltpu.CompilerParams(
            collective_id=0, vmem_limit_bytes=60 * 1024 * 1024),
    )(x2, Wq, Wo, K, V)
    return out.reshape(1, SQ, D)


# device time: 140283 ns/iter; 3.3417x vs baseline; 3.3417x over previous
import jax
import jax.numpy as jnp
from jax import lax
from jax.experimental import pallas as pl
from jax.experimental.pallas import tpu as pltpu

N_DEV = 16
SQ = 512
SKV = 2048
D = 1024
HQ = 8
DH = 128
KBLK = 512
CH = SQ // N_DEV
SCALE = 0.08838834764831843


def kernel(x, Wq, Wo, K_ext, V_ext):
    x2 = x.reshape(SQ, D)
    K = K_ext.reshape(SKV, HQ, DH)
    V = V_ext.reshape(SKV, HQ, DH)

    def body(x_ref, wq_ref, wo_ref, k_hbm, v_hbm, out_ref,
             q_ref, acc_ref, l_ref, co_ref, cl_ref, ag_ref,
             kh_ref, vh_ref, copy_sems,
             rs_send, rs_recv, ls_send, ls_recv, ag_send, ag_recv):
        my = lax.axis_index("i")
        left = lax.rem(my + N_DEV - 1, N_DEV)
        right = lax.rem(my + 1, N_DEV)

        barrier_sem = pltpu.get_barrier_semaphore()
        pl.semaphore_signal(barrier_sem, inc=1, device_id=(left,),
                            device_id_type=pl.DeviceIdType.MESH)
        pl.semaphore_signal(barrier_sem, inc=1, device_id=(right,),
                            device_id_type=pl.DeviceIdType.MESH)
        pl.semaphore_wait(barrier_sem, 2)

        q_ref[...] = jnp.dot(x_ref[...], wq_ref[...],
                             preferred_element_type=jnp.float32)

        for h in range(HQ):
            ck = pltpu.make_async_copy(k_hbm.at[:, h, :], kh_ref,
                                       copy_sems.at[0])
            cv = pltpu.make_async_copy(v_hbm.at[:, h, :], vh_ref,
                                       copy_sems.at[1])
            ck.start()
            cv.start()
            ck.wait()
            cv.wait()

            qh = q_ref[:, h * DH:(h + 1) * DH]
            l = None
            o = None
            for b in range(SKV // KBLK):
                kb = kh_ref[b * KBLK:(b + 1) * KBLK, :]
                vb = vh_ref[b * KBLK:(b + 1) * KBLK, :]
                s = lax.dot_general(qh, kb, (((1,), (1,)), ((), ())),
                                    preferred_element_type=jnp.float32) * SCALE
                p = jnp.exp(s)
                if b == 0:
                    l = jnp.sum(p, axis=1, keepdims=True)
                    o = jnp.dot(p, vb, preferred_element_type=jnp.float32)
                else:
                    l = l + jnp.sum(p, axis=1, keepdims=True)
                    o = o + jnp.dot(p, vb, preferred_element_type=jnp.float32)
            acc_ref[:, h * DH:(h + 1) * DH] = o
            l_ref[:, h:h + 1] = l

        co_ref[0] = acc_ref[pl.ds(my * CH, CH), :]
        cl_ref[0] = l_ref[pl.ds(my * CH, CH), :]
        for t in range(N_DEV - 1):
            ss = t % 2
            rs = (t + 1) % 2
            r_o = pltpu.make_async_remote_copy(
                src_ref=co_ref.at[ss], dst_ref=co_ref.at[rs],
                send_sem=rs_send.at[ss], recv_sem=rs_recv.at[rs],
                device_id=(right,), device_id_type=pl.DeviceIdType.MESH)
            r_l = pltpu.make_async_remote_copy(
                src_ref=cl_ref.at[ss], dst_ref=cl_ref.at[rs],
                send_sem=ls_send.at[ss], recv_sem=ls_recv.at[rs],
                device_id=(right,), device_id_type=pl.DeviceIdType.MESH)
            r_o.start()
            r_l.start()
            r_o.wait()
            r_l.wait()

            c = lax.rem(my - t - 1 + 2 * N_DEV, N_DEV)
            co_ref[rs] = co_ref[rs] + acc_ref[pl.ds(c * CH, CH), :]
            cl_ref[rs] = cl_ref[rs] + l_ref[pl.ds(c * CH, CH), :]

        fin = (N_DEV - 1) % 2
        for h in range(HQ):
            co_ref[fin, :, h * DH:(h + 1) * DH] = (
                co_ref[fin, :, h * DH:(h + 1) * DH]
                / cl_ref[fin, :, h:h + 1])
        y = jnp.dot(co_ref[fin], wo_ref[...],
                    preferred_element_type=jnp.float32)

        c_own = lax.rem(my + 1, N_DEV)
        ag_ref[0] = y
        out_ref[pl.ds(c_own * CH, CH), :] = y
        for t in range(N_DEV - 1):
            ss = t % 2
            rs = (t + 1) % 2
            r_g = pltpu.make_async_remote_copy(
                src_ref=ag_ref.at[ss], dst_ref=ag_ref.at[rs],
                send_sem=ag_send.at[ss], recv_sem=ag_recv.at[rs],
                device_id=(right,), device_id_type=pl.DeviceIdType.MESH)
            r_g.start()
            r_g.wait()
            c = lax.rem(my - t + 2 * N_DEV, N_DEV)
            out_ref[pl.ds(c * CH, CH), :] = ag_ref[rs]

    out = pl.pallas_call(
        body,
        out_shape=jax.ShapeDtypeStruct((SQ, D), jnp.float32),
        in_specs=[
            pl.BlockSpec(memory_space=pltpu.VMEM),
            pl.BlockSpec(memory_space=pltpu.VMEM),
            pl.BlockSpec(memory_space=pltpu.VMEM),
            pl.BlockSpec(memory_space=pl.ANY),
            pl.BlockSpec(memory_space=pl.ANY),
        ],
        out_specs=pl.BlockSpec(memory_space=pltpu.VMEM),
        scratch_shapes=[
            pltpu.VMEM((SQ, D), jnp.float32),
            pltpu.VMEM((SQ, D), jnp.float32),
            pltpu.VMEM((SQ, HQ), jnp.float32),
            pltpu.VMEM((2, CH, D), jnp.float32),
            pltpu.VMEM((2, CH, HQ), jnp.float32),
            pltpu.VMEM((2, CH, D), jnp.float32),
            pltpu.VMEM((SKV, DH), jnp.float32),
            pltpu.VMEM((SKV, DH), jnp.float32),
            pltpu.SemaphoreType.DMA((2,)),
            pltpu.SemaphoreType.DMA((2,)),
            pltpu.SemaphoreType.DMA((2,)),
            pltpu.SemaphoreType.DMA((2,)),
            pltpu.SemaphoreType.DMA((2,)),
            pltpu.SemaphoreType.DMA((2,)),
            pltpu.SemaphoreType.DMA((2,)),
        ],
        compiler_params=pltpu.CompilerParams(
            collective_id=0, vmem_limit_bytes=60 * 1024 * 1024),
    )(x2, Wq, Wo, K, V)
    return out.reshape(1, SQ, D)


# device time: 89187 ns/iter; 5.2562x vs baseline; 1.5729x over previous
import jax
import jax.numpy as jnp
from jax import lax
from jax.experimental import pallas as pl
from jax.experimental.pallas import tpu as pltpu

N_DEV = 16
SQ = 512
SKV = 2048
D = 1024
HQ = 8
DH = 128
KBLK = 512
CH = SQ // N_DEV
SCALE = 0.08838834764831843
MESH = pl.DeviceIdType.MESH


def kernel(x, Wq, Wo, K_ext, V_ext):
    x2 = x.reshape(SQ, D)
    K = K_ext.reshape(SKV, HQ, DH)
    V = V_ext.reshape(SKV, HQ, DH)

    def body(x_ref, wq_ref, wo_ref, k_hbm, v_hbm, out_ref,
             q_ref, acc_ref, l_ref, rs_buf, rl_buf, ag_buf, fin_ref,
             kh_ref, vh_ref, copy_sems,
             rs_osend, rs_orecv, rs_lsend, rs_lrecv, ag_send, ag_recv):
        my = lax.axis_index("i")

        barrier_sem = pltpu.get_barrier_semaphore()
        for j in range(1, N_DEV):
            tgt = lax.rem(my + j, N_DEV)
            pl.semaphore_signal(barrier_sem, inc=1, device_id=(tgt,),
                                device_id_type=MESH)
        pl.semaphore_wait(barrier_sem, N_DEV - 1)

        q_ref[...] = jnp.dot(x_ref[...], wq_ref[...],
                             preferred_element_type=jnp.float32)

        for h in range(HQ):
            ck = pltpu.make_async_copy(k_hbm.at[:, h, :], kh_ref,
                                       copy_sems.at[0])
            cv = pltpu.make_async_copy(v_hbm.at[:, h, :], vh_ref,
                                       copy_sems.at[1])
            ck.start()
            cv.start()
            ck.wait()
            cv.wait()

            qh = q_ref[:, h * DH:(h + 1) * DH]
            l = None
            o = None
            for b in range(SKV // KBLK):
                kb = kh_ref[b * KBLK:(b + 1) * KBLK, :]
                vb = vh_ref[b * KBLK:(b + 1) * KBLK, :]
                s = lax.dot_general(qh, kb, (((1,), (1,)), ((), ())),
                                    preferred_element_type=jnp.float32) * SCALE
                p = jnp.exp(s)
                if b == 0:
                    l = jnp.sum(p, axis=1, keepdims=True)
                    o = jnp.dot(p, vb, preferred_element_type=jnp.float32)
                else:
                    l = l + jnp.sum(p, axis=1, keepdims=True)
                    o = o + jnp.dot(p, vb, preferred_element_type=jnp.float32)
            acc_ref[:, h * DH:(h + 1) * DH] = o
            l_ref[:, h:h + 1] = l

        send_descs = []

        for j in range(1, N_DEV):
            tgt = lax.rem(my + j, N_DEV)
            slot = N_DEV - j
            ro = pltpu.make_async_remote_copy(
                src_ref=acc_ref.at[pl.ds(tgt * CH, CH), :],
                dst_ref=rs_buf.at[slot],
                send_sem=rs_osend.at[j], recv_sem=rs_orecv.at[slot],
                device_id=(tgt,), device_id_type=MESH)
            rl = pltpu.make_async_remote_copy(
                src_ref=l_ref.at[pl.ds(tgt * CH, CH), :],
                dst_ref=rl_buf.at[slot],
                send_sem=rs_lsend.at[j], recv_sem=rs_lrecv.at[slot],
                device_id=(tgt,), device_id_type=MESH)
            ro.start()
            rl.start()
            send_descs.append(ro)
            send_descs.append(rl)

        tot_o = acc_ref[pl.ds(my * CH, CH), :]
        tot_l = l_ref[pl.ds(my * CH, CH), :]
        for j in range(1, N_DEV):
            slot = N_DEV - j
            wo_d = pltpu.make_async_remote_copy(
                src_ref=rs_buf.at[slot], dst_ref=rs_buf.at[slot],
                send_sem=rs_osend.at[j], recv_sem=rs_orecv.at[slot],
                device_id=(my,), device_id_type=MESH)
            wl_d = pltpu.make_async_remote_copy(
                src_ref=rl_buf.at[slot], dst_ref=rl_buf.at[slot],
                send_sem=rs_lsend.at[j], recv_sem=rs_lrecv.at[slot],
                device_id=(my,), device_id_type=MESH)
            wo_d.wait_recv()
            wl_d.wait_recv()
            tot_o = tot_o + rs_buf[slot]
            tot_l = tot_l + rl_buf[slot]

        fin_ref[...] = tot_o
        for h in range(HQ):
            fin_ref[:, h * DH:(h + 1) * DH] = (
                fin_ref[:, h * DH:(h + 1) * DH] / tot_l[:, h:h + 1])
        y = jnp.dot(fin_ref[...], wo_ref[...],
                    preferred_element_type=jnp.float32)
        out_ref[pl.ds(my * CH, CH), :] = y
        fin_ref[...] = y

        for j in range(1, N_DEV):
            tgt = lax.rem(my + j, N_DEV)
            slot = N_DEV - j
            g = pltpu.make_async_remote_copy(
                src_ref=fin_ref,
                dst_ref=ag_buf.at[slot],
                send_sem=ag_send.at[j], recv_sem=ag_recv.at[slot],
                device_id=(tgt,), device_id_type=MESH)
            g.start()
            send_descs.append(g)

        for j in range(1, N_DEV):
            slot = N_DEV - j
            src_dev = lax.rem(my - j + N_DEV, N_DEV)
            w = pltpu.make_async_remote_copy(
                src_ref=ag_buf.at[slot], dst_ref=ag_buf.at[slot],
                send_sem=ag_send.at[j], recv_sem=ag_recv.at[slot],
                device_id=(my,), device_id_type=MESH)
            w.wait_recv()
            out_ref[pl.ds(src_dev * CH, CH), :] = ag_buf[slot]

        for d in send_descs:
            d.wait_send()

    out = pl.pallas_call(
        body,
        out_shape=jax.ShapeDtypeStruct((SQ, D), jnp.float32),
        in_specs=[
            pl.BlockSpec(memory_space=pltpu.VMEM),
            pl.BlockSpec(memory_space=pltpu.VMEM),
            pl.BlockSpec(memory_space=pltpu.VMEM),
            pl.BlockSpec(memory_space=pl.ANY),
            pl.BlockSpec(memory_space=pl.ANY),
        ],
        out_specs=pl.BlockSpec(memory_space=pltpu.VMEM),
        scratch_shapes=[
            pltpu.VMEM((SQ, D), jnp.float32),
            pltpu.VMEM((SQ, D), jnp.float32),
            pltpu.VMEM((SQ, HQ), jnp.float32),
            pltpu.VMEM((N_DEV, CH, D), jnp.float32),
            pltpu.VMEM((N_DEV, CH, HQ), jnp.float32),
            pltpu.VMEM((N_DEV, CH, D), jnp.float32),
            pltpu.VMEM((CH, D), jnp.float32),
            pltpu.VMEM((SKV, DH), jnp.float32),
            pltpu.VMEM((SKV, DH), jnp.float32),
            pltpu.SemaphoreType.DMA((2,)),
            pltpu.SemaphoreType.DMA((N_DEV,)),
            pltpu.SemaphoreType.DMA((N_DEV,)),
            pltpu.SemaphoreType.DMA((N_DEV,)),
            pltpu.SemaphoreType.DMA((N_DEV,)),
            pltpu.SemaphoreType.DMA((N_DEV,)),
            pltpu.SemaphoreType.DMA((N_DEV,)),
        ],
        compiler_params=pltpu.CompilerParams(
            collective_id=0, vmem_limit_bytes=60 * 1024 * 1024),
    )(x2, Wq, Wo, K, V)
    return out.reshape(1, SQ, D)


# device time: 88704 ns/iter; 5.2848x vs baseline; 1.0054x over previous
import jax
import jax.numpy as jnp
from jax import lax
from jax.experimental import pallas as pl
from jax.experimental.pallas import tpu as pltpu

N_DEV = 16
SQ = 512
SKV = 2048
D = 1024
HQ = 8
DH = 128
KBLK = 512
CH = SQ // N_DEV
SCALE = 0.08838834764831843
MESH = pl.DeviceIdType.MESH


def kernel(x, Wq, Wo, K_ext, V_ext):
    x2 = x.reshape(SQ, D)
    K = K_ext.reshape(SKV, HQ, DH)
    V = V_ext.reshape(SKV, HQ, DH)

    def body(x_ref, wq_ref, wo_ref, k_hbm, v_hbm, out_ref,
             q_ref, acc_ref, l_ref, rs_buf, rl_buf, ag_buf, fin_ref,
             kh_ref, vh_ref, copy_sems,
             rs_osend, rs_orecv, rs_lsend, rs_lrecv, ag_send, ag_recv):
        my = lax.axis_index("i")

        barrier_sem = pltpu.get_barrier_semaphore()
        for j in range(1, N_DEV):
            tgt = lax.rem(my + j, N_DEV)
            pl.semaphore_signal(barrier_sem, inc=1, device_id=(tgt,),
                                device_id_type=MESH)
        pl.semaphore_wait(barrier_sem, N_DEV - 1)

        q_ref[...] = (jnp.dot(x_ref[...].astype(jnp.bfloat16),
                              wq_ref[...].astype(jnp.bfloat16),
                              preferred_element_type=jnp.float32)
                      * (SCALE * 1.4426950408889634)).astype(jnp.bfloat16)

        for h in range(HQ):
            ck = pltpu.make_async_copy(k_hbm.at[:, h, :], kh_ref,
                                       copy_sems.at[0])
            cv = pltpu.make_async_copy(v_hbm.at[:, h, :], vh_ref,
                                       copy_sems.at[1])
            ck.start()
            cv.start()
            ck.wait()
            cv.wait()

            qh = q_ref[:, h * DH:(h + 1) * DH]
            l = None
            o = None
            for b in range(SKV // KBLK):
                kb = kh_ref[b * KBLK:(b + 1) * KBLK, :].astype(jnp.bfloat16)
                vb = vh_ref[b * KBLK:(b + 1) * KBLK, :].astype(jnp.bfloat16)
                s = lax.dot_general(qh, kb, (((1,), (1,)), ((), ())),
                                    preferred_element_type=jnp.float32)
                p = jnp.exp2(s)
                pb = p.astype(jnp.bfloat16)
                if b == 0:
                    l = jnp.sum(p, axis=1, keepdims=True)
                    o = jnp.dot(pb, vb, preferred_element_type=jnp.float32)
                else:
                    l = l + jnp.sum(p, axis=1, keepdims=True)
                    o = o + jnp.dot(pb, vb, preferred_element_type=jnp.float32)
            acc_ref[:, h * DH:(h + 1) * DH] = o
            l_ref[:, h:h + 1] = l

        send_descs = []

        for j in range(1, N_DEV):
            tgt = lax.rem(my + j, N_DEV)
            slot = N_DEV - j
            ro = pltpu.make_async_remote_copy(
                src_ref=acc_ref.at[pl.ds(tgt * CH, CH), :],
                dst_ref=rs_buf.at[slot],
                send_sem=rs_osend.at[j], recv_sem=rs_orecv.at[slot],
                device_id=(tgt,), device_id_type=MESH)
            rl = pltpu.make_async_remote_copy(
                src_ref=l_ref.at[pl.ds(tgt * CH, CH), :],
                dst_ref=rl_buf.at[slot],
                send_sem=rs_lsend.at[j], recv_sem=rs_lrecv.at[slot],
                device_id=(tgt,), device_id_type=MESH)
            ro.start()
            rl.start()
            send_descs.append(ro)
            send_descs.append(rl)

        tot_o = acc_ref[pl.ds(my * CH, CH), :]
        tot_l = l_ref[pl.ds(my * CH, CH), :]
        for j in range(1, N_DEV):
            slot = N_DEV - j
            wo_d = pltpu.make_async_remote_copy(
                src_ref=rs_buf.at[slot], dst_ref=rs_buf.at[slot],
                send_sem=rs_osend.at[j], recv_sem=rs_orecv.at[slot],
                device_id=(my,), device_id_type=MESH)
            wl_d = pltpu.make_async_remote_copy(
                src_ref=rl_buf.at[slot], dst_ref=rl_buf.at[slot],
                send_sem=rs_lsend.at[j], recv_sem=rs_lrecv.at[slot],
                device_id=(my,), device_id_type=MESH)
            wo_d.wait_recv()
            wl_d.wait_recv()
            tot_o = tot_o + rs_buf[slot]
            tot_l = tot_l + rl_buf[slot]

        fin_ref[...] = tot_o
        for h in range(HQ):
            fin_ref[:, h * DH:(h + 1) * DH] = (
                fin_ref[:, h * DH:(h + 1) * DH] / tot_l[:, h:h + 1])
        y = jnp.dot(fin_ref[...].astype(jnp.bfloat16),
                    wo_ref[...].astype(jnp.bfloat16),
                    preferred_element_type=jnp.float32)
        out_ref[pl.ds(my * CH, CH), :] = y
        fin_ref[...] = y

        for j in range(1, N_DEV):
            tgt = lax.rem(my + j, N_DEV)
            slot = N_DEV - j
            g = pltpu.make_async_remote_copy(
                src_ref=fin_ref,
                dst_ref=ag_buf.at[slot],
                send_sem=ag_send.at[j], recv_sem=ag_recv.at[slot],
                device_id=(tgt,), device_id_type=MESH)
            g.start()
            send_descs.append(g)

        for j in range(1, N_DEV):
            slot = N_DEV - j
            src_dev = lax.rem(my - j + N_DEV, N_DEV)
            w = pltpu.make_async_remote_copy(
                src_ref=ag_buf.at[slot], dst_ref=ag_buf.at[slot],
                send_sem=ag_send.at[j], recv_sem=ag_recv.at[slot],
                device_id=(my,), device_id_type=MESH)
            w.wait_recv()
            out_ref[pl.ds(src_dev * CH, CH), :] = ag_buf[slot]

        for d in send_descs:
            d.wait_send()

    out = pl.pallas_call(
        body,
        out_shape=jax.ShapeDtypeStruct((SQ, D), jnp.float32),
        in_specs=[
            pl.BlockSpec(memory_space=pltpu.VMEM),
            pl.BlockSpec(memory_space=pltpu.VMEM),
            pl.BlockSpec(memory_space=pltpu.VMEM),
            pl.BlockSpec(memory_space=pl.ANY),
            pl.BlockSpec(memory_space=pl.ANY),
        ],
        out_specs=pl.BlockSpec(memory_space=pltpu.VMEM),
        scratch_shapes=[
            pltpu.VMEM((SQ, D), jnp.bfloat16),
            pltpu.VMEM((SQ, D), jnp.float32),
            pltpu.VMEM((SQ, HQ), jnp.float32),
            pltpu.VMEM((N_DEV, CH, D), jnp.float32),
            pltpu.VMEM((N_DEV, CH, HQ), jnp.float32),
            pltpu.VMEM((N_DEV, CH, D), jnp.float32),
            pltpu.VMEM((CH, D), jnp.float32),
            pltpu.VMEM((SKV, DH), jnp.float32),
            pltpu.VMEM((SKV, DH), jnp.float32),
            pltpu.SemaphoreType.DMA((2,)),
            pltpu.SemaphoreType.DMA((N_DEV,)),
            pltpu.SemaphoreType.DMA((N_DEV,)),
            pltpu.SemaphoreType.DMA((N_DEV,)),
            pltpu.SemaphoreType.DMA((N_DEV,)),
            pltpu.SemaphoreType.DMA((N_DEV,)),
            pltpu.SemaphoreType.DMA((N_DEV,)),
        ],
        compiler_params=pltpu.CompilerParams(
            collective_id=0, vmem_limit_bytes=60 * 1024 * 1024),
    )(x2, Wq, Wo, K, V)
    return out.reshape(1, SQ, D)
